# baseline (speedup 1.0000x reference)
import functools

import jax
import jax.numpy as jnp
from jax import lax
from jax.experimental import pallas as pl
from jax.experimental.pallas import tpu as pltpu
from jax.experimental.pallas import tpu_sc as plsc

E = 16
D = 1024
HID = 4096
T = 4096
CAP = 512
NR = 8
BH = 512
NH = HID // BH

_HI_BITS = 0x3F800001


def _scores_body(x_ref, gw_ref, out_ref, aux_ref):
    logits = lax.dot_general(x_ref[...], gw_ref[...], (((1,), (1,)), ((), ())))
    scores = jax.nn.sigmoid(logits)
    out_ref[...] = scores
    s_int = lax.bitcast_convert_type(scores, jnp.int32)

    def bs_step(_, carry):
        lo, hi = carry
        mid = (lo + hi) // 2
        cnt = jnp.sum((s_int >= mid).astype(jnp.int32), axis=0, keepdims=True)
        pred = cnt >= CAP
        return (jnp.where(pred, mid, lo), jnp.where(pred, hi, mid))

    lo0 = jnp.zeros((1, E), jnp.int32)
    hi0 = jnp.full((1, E), _HI_BITS, jnp.int32)
    lo, _ = lax.fori_loop(0, 31, bs_step, (lo0, hi0))
    cnt_gt = jnp.sum((s_int >= lo + 1).astype(jnp.int32), axis=0, keepdims=True)
    need_eq = CAP - cnt_gt
    aux_ref[...] = jnp.concatenate([lo, need_eq, lo, need_eq], axis=0)


def _scores(x2d, gate_weight):
    return pl.pallas_call(
        _scores_body,
        out_shape=[jax.ShapeDtypeStruct((T, E), jnp.float32),
                   jax.ShapeDtypeStruct((4, E), jnp.int32)],
    )(x2d, gate_weight)


def _topk_body(aux_ref, s_ref, ids_ref, wts_ref):
    e = pl.program_id(0)
    s = s_ref[0]
    s_int = lax.bitcast_convert_type(s, jnp.int32)
    thr = aux_ref[0, e]
    need_eq = aux_ref[1, e].astype(jnp.float32)
    m_gt = s_int >= thr + 1
    m_eq = s_int == thr

    cio = lax.broadcasted_iota(jnp.int32, (512, 512), 0)
    jio = lax.broadcasted_iota(jnp.int32, (512, 512), 1)
    tri = (cio <= jio).astype(jnp.float32)
    rio = lax.broadcasted_iota(jnp.int32, (NR, NR), 0)
    rjo = lax.broadcasted_iota(jnp.int32, (NR, NR), 1)
    strict = (rjo < rio).astype(jnp.float32)

    def cumsum2(mf):
        rowcum = lax.dot_general(mf, tri, (((1,), (0,)), ((), ())))
        rowtot = rowcum[:, 511:512]
        carry = lax.dot_general(strict, rowtot, (((1,), (0,)), ((), ())))
        return rowcum + carry

    eqrank = cumsum2(m_eq.astype(jnp.float32))
    m = m_gt | (m_eq & (eqrank <= need_eq))
    mf = m.astype(jnp.float32)
    pm = cumsum2(mf) * mf

    jslot = lax.broadcasted_iota(jnp.int32, (CAP, 512), 0) + 1
    cval = lax.broadcasted_iota(jnp.int32, (CAP, 512), 1).astype(jnp.float32)
    pm_i = pm.astype(jnp.int32)
    ids_acc = jnp.zeros((CAP, 1), jnp.float32)
    wts_acc = jnp.zeros((CAP, 1), jnp.float32)
    for r in range(NR):
        oh = jslot == pm_i[r:r + 1, :]
        ids_acc += jnp.sum(jnp.where(oh, cval + (512.0 * r), 0.0),
                           axis=1, keepdims=True)
        wts_acc += jnp.sum(jnp.where(oh, s[r:r + 1, :], 0.0),
                           axis=1, keepdims=True)
    ids_ref[0] = ids_acc.astype(jnp.int32)
    wts_ref[0] = wts_acc


def _topk(aux, sT3):
    return pl.pallas_call(
        _topk_body,
        grid=(E,),
        in_specs=[pl.BlockSpec(memory_space=pltpu.SMEM),
                  pl.BlockSpec((1, NR, 512), lambda e: (e, 0, 0))],
        out_specs=[pl.BlockSpec((1, CAP, 1), lambda e: (e, 0, 0)),
                   pl.BlockSpec((1, CAP, 1), lambda e: (e, 0, 0))],
        out_shape=[jax.ShapeDtypeStruct((E, CAP, 1), jnp.int32),
                   jax.ShapeDtypeStruct((E, CAP, 1), jnp.float32)],
    )(aux, sT3)


_NW = 32
_RPW = (E * CAP) // _NW
_CH = 32
_NCH = _RPW // _CH


def _gather(x2d, flat_ids):
    mesh = plsc.VectorSubcoreMesh(core_axis_name="c", subcore_axis_name="s")

    @functools.partial(
        pl.kernel,
        out_type=jax.ShapeDtypeStruct((E * CAP, D), jnp.float32),
        mesh=mesh,
        scratch_types=[
            pltpu.VMEM((_RPW,), jnp.int32),
            pltpu.VMEM((_CH, D), jnp.float32),
            pltpu.VMEM((_CH, D), jnp.float32),
            pltpu.SemaphoreType.DMA,
            pltpu.SemaphoreType.DMA,
        ],
    )
    def k(x_hbm, ids_hbm, out_hbm, idx_v, buf0, buf1, sem0, sem1):
        wid = lax.axis_index("s") * 2 + lax.axis_index("c")
        base = wid * _RPW
        pltpu.sync_copy(ids_hbm.at[pl.ds(base, _RPW)], idx_v)
        bufs = (buf0, buf1)
        sems = (sem0, sem1)
        descs = [None, None]
        descs[0] = pltpu.async_copy(
            x_hbm.at[idx_v.at[pl.ds(0, _CH)]], bufs[0], sems[0])
        for c in range(_NCH):
            nxt = c + 1
            if nxt < _NCH:
                descs[nxt % 2] = pltpu.async_copy(
                    x_hbm.at[idx_v.at[pl.ds(nxt * _CH, _CH)]],
                    bufs[nxt % 2], sems[nxt % 2])
            descs[c % 2].wait()
            pltpu.sync_copy(bufs[c % 2],
                            out_hbm.at[pl.ds(base + c * _CH, _CH)])

    return k(x2d, flat_ids)


DBLK = 256
ND = D // DBLK


def _ffn_body(xg_ref, uw_ref, ub_ref, gw_ref, gb_ref, dw_ref, db_ref,
              wts_ref, ids_ref, out_ref, hbuf_ref, oh_ref):
    e = pl.program_id(0)
    g = pl.program_id(1)
    dn = (((1,), (1,)), ((), ()))

    @pl.when(g < NH)
    def _():
        xb = xg_ref[0].astype(jnp.bfloat16)
        uw = uw_ref[0].astype(jnp.bfloat16)
        gw = gw_ref[0].astype(jnp.bfloat16)
        up = lax.dot_general(xb, uw, dn,
                             preferred_element_type=jnp.float32) + ub_ref[0, 0]
        gt = lax.dot_general(xb, gw, dn,
                             preferred_element_type=jnp.float32) + gb_ref[0, 0]
        hb = (gt * jax.nn.sigmoid(gt) * up).astype(jnp.bfloat16)
        hbuf_ref[:, pl.ds(g * BH, BH)] = hb

    @pl.when(g == NH)
    def _():
        tio = lax.broadcasted_iota(jnp.int32, (T, CAP), 0)
        oh_ref[...] = (tio == ids_ref[0]).astype(jnp.bfloat16)

    @pl.when(g >= NH)
    def _():
        dwb = dw_ref[0].astype(jnp.bfloat16)
        part = lax.dot_general(hbuf_ref[...], dwb, dn,
                               preferred_element_type=jnp.float32)
        finalb = ((part + db_ref[0, 0]) * wts_ref[0]).astype(jnp.bfloat16)
        outc = lax.dot_general(oh_ref[...], finalb, (((1,), (0,)), ((), ())),
                               preferred_element_type=jnp.float32)
        dblk = g - NH

        @pl.when(e == 0)
        def _():
            out_ref[:, pl.ds(dblk * DBLK, DBLK)] = outc

        @pl.when(e > 0)
        def _():
            out_ref[:, pl.ds(dblk * DBLK, DBLK)] += outc


def _ffn(xg4, up_w, up_b3, gateproj_w, gp_b3, down_w, db4, wts3, ids_row3):
    return pl.pallas_call(
        _ffn_body,
        grid=(E, NH + ND),
        in_specs=[
            pl.BlockSpec((1, CAP, D), lambda e, g: (e, 0, 0)),
            pl.BlockSpec((1, BH, D),
                         lambda e, g: (e, jnp.minimum(g, NH - 1), 0)),
            pl.BlockSpec((1, 1, 1, BH),
                         lambda e, g: (e, jnp.minimum(g, NH - 1), 0, 0)),
            pl.BlockSpec((1, BH, D),
                         lambda e, g: (e, jnp.minimum(g, NH - 1), 0)),
            pl.BlockSpec((1, 1, 1, BH),
                         lambda e, g: (e, jnp.minimum(g, NH - 1), 0, 0)),
            pl.BlockSpec((1, DBLK, HID),
                         lambda e, g: (e, jnp.maximum(g - NH, 0), 0)),
            pl.BlockSpec((1, 1, 1, DBLK),
                         lambda e, g: (e, jnp.maximum(g - NH, 0), 0, 0)),
            pl.BlockSpec((1, CAP, 1), lambda e, g: (e, 0, 0)),
            pl.BlockSpec((1, 1, CAP), lambda e, g: (e, 0, 0)),
        ],
        out_specs=pl.BlockSpec((T, D), lambda e, g: (0, 0)),
        out_shape=jax.ShapeDtypeStruct((T, D), jnp.float32),
        scratch_shapes=[pltpu.VMEM((CAP, HID), jnp.bfloat16),
                        pltpu.VMEM((T, CAP), jnp.bfloat16)],
    )(xg4, up_w, up_b3, gateproj_w, gp_b3, down_w, db4, wts3, ids_row3)


def kernel(hidden_states, gate_weight, up_w, up_b, gateproj_w, gateproj_b,
           down_w, down_b):
    b, s, d = hidden_states.shape
    x2d = hidden_states.reshape(b * s, d)
    scores, aux = _scores(x2d, gate_weight)
    sT3 = scores.T.reshape(E, NR, 512)
    ids3, wts3 = _topk(aux, sT3)
    flat_ids = ids3.reshape(E * CAP)
    xg = _gather(x2d, flat_ids)
    out = _ffn(
        xg.reshape(E, CAP, D),
        up_w, up_b.reshape(E, NH, 1, BH),
        gateproj_w, gateproj_b.reshape(E, NH, 1, BH),
        down_w, down_b.reshape(E, ND, 1, DBLK),
        wts3, ids3.reshape(E, 1, CAP),
    )
    return out.reshape(b, s, d)


# speedup vs baseline: 1.2835x; 1.2835x over previous
import functools

import jax
import jax.numpy as jnp
from jax import lax
from jax.experimental import pallas as pl
from jax.experimental.pallas import tpu as pltpu
from jax.experimental.pallas import tpu_sc as plsc

E = 16
D = 1024
HID = 4096
T = 4096
CAP = 512
NR = 8
BH = 512
NH = HID // BH

_HI_BITS = 0x3F800001


def _scores_body(x_ref, gw_ref, out_ref, aux_ref):
    logits = lax.dot_general(x_ref[...], gw_ref[...], (((1,), (1,)), ((), ())))
    scores = jax.nn.sigmoid(logits)
    out_ref[...] = scores
    s_int = lax.bitcast_convert_type(scores, jnp.int32)

    def bs_step(_, carry):
        lo, hi = carry
        mid = (lo + hi) // 2
        cnt = jnp.sum((s_int >= mid).astype(jnp.int32), axis=0, keepdims=True)
        pred = cnt >= CAP
        return (jnp.where(pred, mid, lo), jnp.where(pred, hi, mid))

    lo0 = jnp.zeros((1, E), jnp.int32)
    hi0 = jnp.full((1, E), _HI_BITS, jnp.int32)
    lo, _ = lax.fori_loop(0, 31, bs_step, (lo0, hi0))
    cnt_gt = jnp.sum((s_int >= lo + 1).astype(jnp.int32), axis=0, keepdims=True)
    need_eq = CAP - cnt_gt
    aux_ref[...] = jnp.concatenate([lo, need_eq, lo, need_eq], axis=0)


def _scores(x2d, gate_weight):
    return pl.pallas_call(
        _scores_body,
        out_shape=[jax.ShapeDtypeStruct((T, E), jnp.float32),
                   jax.ShapeDtypeStruct((4, E), jnp.int32)],
    )(x2d, gate_weight)


def _topk_body(aux_ref, s_ref, ids_ref, wts_ref):
    e = pl.program_id(0)
    s = s_ref[0]
    s_int = lax.bitcast_convert_type(s, jnp.int32)
    thr = aux_ref[0, e]
    need_eq = aux_ref[1, e].astype(jnp.float32)
    m_gt = s_int >= thr + 1
    m_eq = s_int == thr

    cio = lax.broadcasted_iota(jnp.int32, (512, 512), 0)
    jio = lax.broadcasted_iota(jnp.int32, (512, 512), 1)
    tri = (cio <= jio).astype(jnp.float32)
    rio = lax.broadcasted_iota(jnp.int32, (NR, NR), 0)
    rjo = lax.broadcasted_iota(jnp.int32, (NR, NR), 1)
    strict = (rjo < rio).astype(jnp.float32)

    def cumsum2(mf):
        rowcum = lax.dot_general(mf, tri, (((1,), (0,)), ((), ())))
        rowtot = rowcum[:, 511:512]
        carry = lax.dot_general(strict, rowtot, (((1,), (0,)), ((), ())))
        return rowcum + carry

    eqrank = cumsum2(m_eq.astype(jnp.float32))
    m = m_gt | (m_eq & (eqrank <= need_eq))
    mf = m.astype(jnp.float32)
    pm = cumsum2(mf) * mf

    jslot = lax.broadcasted_iota(jnp.int32, (CAP, 512), 0) + 1
    cval = lax.broadcasted_iota(jnp.int32, (CAP, 512), 1).astype(jnp.float32)
    pm_i = pm.astype(jnp.int32)
    ids_acc = jnp.zeros((CAP, 1), jnp.float32)
    wts_acc = jnp.zeros((CAP, 1), jnp.float32)
    for r in range(NR):
        oh = jslot == pm_i[r:r + 1, :]
        ids_acc += jnp.sum(jnp.where(oh, cval + (512.0 * r), 0.0),
                           axis=1, keepdims=True)
        wts_acc += jnp.sum(jnp.where(oh, s[r:r + 1, :], 0.0),
                           axis=1, keepdims=True)
    ids_ref[0] = ids_acc.astype(jnp.int32)
    wts_ref[0] = wts_acc


def _topk(aux, sT3):
    return pl.pallas_call(
        _topk_body,
        grid=(E,),
        in_specs=[pl.BlockSpec(memory_space=pltpu.SMEM),
                  pl.BlockSpec((1, NR, 512), lambda e: (e, 0, 0))],
        out_specs=[pl.BlockSpec((1, CAP, 1), lambda e: (e, 0, 0)),
                   pl.BlockSpec((1, CAP, 1), lambda e: (e, 0, 0))],
        out_shape=[jax.ShapeDtypeStruct((E, CAP, 1), jnp.int32),
                   jax.ShapeDtypeStruct((E, CAP, 1), jnp.float32)],
    )(aux, sT3)


_NW = 32
_RPW = (E * CAP) // _NW
_CH = 32
_NCH = _RPW // _CH


def _gather(x2d, flat_ids):
    mesh = plsc.VectorSubcoreMesh(core_axis_name="c", subcore_axis_name="s")

    @functools.partial(
        pl.kernel,
        out_type=jax.ShapeDtypeStruct((E * CAP, D), jnp.float32),
        mesh=mesh,
        scratch_types=[
            pltpu.VMEM((_RPW,), jnp.int32),
            pltpu.VMEM((_CH, D), jnp.float32),
            pltpu.VMEM((_CH, D), jnp.float32),
            pltpu.SemaphoreType.DMA,
            pltpu.SemaphoreType.DMA,
        ],
    )
    def k(x_hbm, ids_hbm, out_hbm, idx_v, buf0, buf1, sem0, sem1):
        wid = lax.axis_index("s") * 2 + lax.axis_index("c")
        base = wid * _RPW
        pltpu.sync_copy(ids_hbm.at[pl.ds(base, _RPW)], idx_v)
        bufs = (buf0, buf1)
        sems = (sem0, sem1)
        descs = [None, None]
        descs[0] = pltpu.async_copy(
            x_hbm.at[idx_v.at[pl.ds(0, _CH)]], bufs[0], sems[0])
        for c in range(_NCH):
            nxt = c + 1
            if nxt < _NCH:
                descs[nxt % 2] = pltpu.async_copy(
                    x_hbm.at[idx_v.at[pl.ds(nxt * _CH, _CH)]],
                    bufs[nxt % 2], sems[nxt % 2])
            descs[c % 2].wait()
            pltpu.sync_copy(bufs[c % 2],
                            out_hbm.at[pl.ds(base + c * _CH, _CH)])

    return k(x2d, flat_ids)


def _ffn_body(xg_ref, uw_ref, ub_ref, gw_ref, gb_ref, dw_ref, db_ref,
              wts_ref, ids_ref, out_ref, acc_ref):
    e = pl.program_id(0)
    h = pl.program_id(1)
    dn = (((1,), (1,)), ((), ()))
    xb = xg_ref[0].astype(jnp.bfloat16)
    uw = uw_ref[0].astype(jnp.bfloat16)
    gw = gw_ref[0].astype(jnp.bfloat16)
    up = lax.dot_general(xb, uw, dn,
                         preferred_element_type=jnp.float32) + ub_ref[0, 0]
    gt = lax.dot_general(xb, gw, dn,
                         preferred_element_type=jnp.float32) + gb_ref[0, 0]
    hb = (gt * jax.nn.sigmoid(gt) * up).astype(jnp.bfloat16)
    dw = dw_ref[0].astype(jnp.bfloat16)
    contrib = lax.dot_general(hb, dw, dn,
                              preferred_element_type=jnp.float32)

    @pl.when(h == 0)
    def _():
        acc_ref[...] = contrib

    @pl.when(h > 0)
    def _():
        acc_ref[...] += contrib

    @pl.when(h == NH - 1)
    def _():
        final = (acc_ref[...] + db_ref[0]) * wts_ref[0]
        tio = lax.broadcasted_iota(jnp.int32, (T, CAP), 0)
        oh = (tio == ids_ref[0]).astype(jnp.bfloat16)
        outc = lax.dot_general(oh, final.astype(jnp.bfloat16),
                               (((1,), (0,)), ((), ())),
                               preferred_element_type=jnp.float32)

        @pl.when(e == 0)
        def _():
            out_ref[...] = outc

        @pl.when(e > 0)
        def _():
            out_ref[...] += outc


def _ffn(xg4, up_w, up_b3, gateproj_w, gp_b3, down_w, db3, wts3, ids_row3):
    return pl.pallas_call(
        _ffn_body,
        grid=(E, NH),
        in_specs=[
            pl.BlockSpec((1, CAP, D), lambda e, h: (e, 0, 0)),
            pl.BlockSpec((1, BH, D), lambda e, h: (e, h, 0)),
            pl.BlockSpec((1, 1, 1, BH), lambda e, h: (e, h, 0, 0)),
            pl.BlockSpec((1, BH, D), lambda e, h: (e, h, 0)),
            pl.BlockSpec((1, 1, 1, BH), lambda e, h: (e, h, 0, 0)),
            pl.BlockSpec((1, D, BH), lambda e, h: (e, 0, h)),
            pl.BlockSpec((1, 1, D), lambda e, h: (e, 0, 0)),
            pl.BlockSpec((1, CAP, 1), lambda e, h: (e, 0, 0)),
            pl.BlockSpec((1, 1, CAP), lambda e, h: (e, 0, 0)),
        ],
        out_specs=pl.BlockSpec((T, D), lambda e, h: (0, 0)),
        out_shape=jax.ShapeDtypeStruct((T, D), jnp.float32),
        scratch_shapes=[pltpu.VMEM((CAP, D), jnp.float32)],
    )(xg4, up_w, up_b3, gateproj_w, gp_b3, down_w, db3, wts3, ids_row3)


def kernel(hidden_states, gate_weight, up_w, up_b, gateproj_w, gateproj_b,
           down_w, down_b):
    b, s, d = hidden_states.shape
    x2d = hidden_states.reshape(b * s, d)
    scores, aux = _scores(x2d, gate_weight)
    sT3 = scores.T.reshape(E, NR, 512)
    ids3, wts3 = _topk(aux, sT3)
    flat_ids = ids3.reshape(E * CAP)
    xg = _gather(x2d, flat_ids)
    out = _ffn(
        xg.reshape(E, CAP, D),
        up_w, up_b.reshape(E, NH, 1, BH),
        gateproj_w, gateproj_b.reshape(E, NH, 1, BH),
        down_w, down_b.reshape(E, 1, D),
        wts3, ids3.reshape(E, 1, CAP),
    )
    return out.reshape(b, s, d)


# speedup vs baseline: 1.2849x; 1.0011x over previous
import functools

import jax
import jax.numpy as jnp
from jax import lax
from jax.experimental import pallas as pl
from jax.experimental.pallas import tpu as pltpu
from jax.experimental.pallas import tpu_sc as plsc

E = 16
D = 1024
HID = 4096
T = 4096
CAP = 512
NR = 8
BH = 512
NH = HID // BH

_HI_BITS = 0x3F800001


def _scores_body(x_ref, gw_ref, out_ref, aux_ref):
    logits = lax.dot_general(x_ref[...], gw_ref[...], (((1,), (1,)), ((), ())))
    scores = jax.nn.sigmoid(logits)
    out_ref[...] = scores
    s_int = lax.bitcast_convert_type(scores, jnp.int32)

    def bs_step(_, carry):
        lo, hi = carry
        mid = (lo + hi) // 2
        cnt = jnp.sum((s_int >= mid).astype(jnp.int32), axis=0, keepdims=True)
        pred = cnt >= CAP
        return (jnp.where(pred, mid, lo), jnp.where(pred, hi, mid))

    lo0 = jnp.zeros((1, E), jnp.int32)
    hi0 = jnp.full((1, E), _HI_BITS, jnp.int32)
    lo, _ = lax.fori_loop(0, 31, bs_step, (lo0, hi0))
    cnt_gt = jnp.sum((s_int >= lo + 1).astype(jnp.int32), axis=0, keepdims=True)
    need_eq = CAP - cnt_gt
    aux_ref[...] = jnp.concatenate([lo, need_eq, lo, need_eq], axis=0)


def _scores(x2d, gate_weight):
    return pl.pallas_call(
        _scores_body,
        out_shape=[jax.ShapeDtypeStruct((T, E), jnp.float32),
                   jax.ShapeDtypeStruct((4, E), jnp.int32)],
    )(x2d, gate_weight)


def _topk_body(aux_ref, s_ref, ids_ref, wts_ref):
    e = pl.program_id(0)
    s = s_ref[0]
    s_int = lax.bitcast_convert_type(s, jnp.int32)
    thr = aux_ref[0, e]
    need_eq = aux_ref[1, e].astype(jnp.float32)
    m_gt = s_int >= thr + 1
    m_eq = s_int == thr

    cio = lax.broadcasted_iota(jnp.int32, (512, 512), 0)
    jio = lax.broadcasted_iota(jnp.int32, (512, 512), 1)
    tri = (cio <= jio).astype(jnp.float32)
    rio = lax.broadcasted_iota(jnp.int32, (NR, NR), 0)
    rjo = lax.broadcasted_iota(jnp.int32, (NR, NR), 1)
    strict = (rjo < rio).astype(jnp.float32)

    def cumsum2(mf):
        rowcum = lax.dot_general(mf, tri, (((1,), (0,)), ((), ())))
        rowtot = rowcum[:, 511:512]
        carry = lax.dot_general(strict, rowtot, (((1,), (0,)), ((), ())))
        return rowcum + carry

    eqrank = cumsum2(m_eq.astype(jnp.float32))
    m = m_gt | (m_eq & (eqrank <= need_eq))
    mf = m.astype(jnp.float32)
    pm = cumsum2(mf) * mf

    jslot = lax.broadcasted_iota(jnp.int32, (CAP, 512), 0) + 1
    cval = lax.broadcasted_iota(jnp.int32, (CAP, 512), 1).astype(jnp.float32)
    pm_i = pm.astype(jnp.int32)
    ids_acc = jnp.zeros((CAP, 1), jnp.float32)
    wts_acc = jnp.zeros((CAP, 1), jnp.float32)
    for r in range(NR):
        oh = jslot == pm_i[r:r + 1, :]
        ids_acc += jnp.sum(jnp.where(oh, cval + (512.0 * r), 0.0),
                           axis=1, keepdims=True)
        wts_acc += jnp.sum(jnp.where(oh, s[r:r + 1, :], 0.0),
                           axis=1, keepdims=True)
    ids_ref[0] = ids_acc.astype(jnp.int32)
    wts_ref[0] = wts_acc


def _topk(aux, sT3):
    return pl.pallas_call(
        _topk_body,
        grid=(E,),
        in_specs=[pl.BlockSpec(memory_space=pltpu.SMEM),
                  pl.BlockSpec((1, NR, 512), lambda e: (e, 0, 0))],
        out_specs=[pl.BlockSpec((1, CAP, 1), lambda e: (e, 0, 0)),
                   pl.BlockSpec((1, CAP, 1), lambda e: (e, 0, 0))],
        out_shape=[jax.ShapeDtypeStruct((E, CAP, 1), jnp.int32),
                   jax.ShapeDtypeStruct((E, CAP, 1), jnp.float32)],
    )(aux, sT3)


_NW = 32
_RPW = (E * CAP) // _NW
_CH = 32
_NCH = _RPW // _CH


def _gather(x2d, flat_ids):
    mesh = plsc.VectorSubcoreMesh(core_axis_name="c", subcore_axis_name="s")

    @functools.partial(
        pl.kernel,
        out_type=jax.ShapeDtypeStruct((E * CAP, D), jnp.float32),
        mesh=mesh,
        scratch_types=[
            pltpu.VMEM((_RPW,), jnp.int32),
            pltpu.VMEM((_CH, D), jnp.float32),
            pltpu.VMEM((_CH, D), jnp.float32),
            pltpu.SemaphoreType.DMA,
            pltpu.SemaphoreType.DMA,
        ],
    )
    def k(x_hbm, ids_hbm, out_hbm, idx_v, buf0, buf1, sem0, sem1):
        wid = lax.axis_index("s") * 2 + lax.axis_index("c")
        base = wid * _RPW
        pltpu.sync_copy(ids_hbm.at[pl.ds(base, _RPW)], idx_v)
        bufs = (buf0, buf1)
        sems = (sem0, sem1)
        descs = [None, None]
        descs[0] = pltpu.async_copy(
            x_hbm.at[idx_v.at[pl.ds(0, _CH)]], bufs[0], sems[0])
        for c in range(_NCH):
            nxt = c + 1
            if nxt < _NCH:
                descs[nxt % 2] = pltpu.async_copy(
                    x_hbm.at[idx_v.at[pl.ds(nxt * _CH, _CH)]],
                    bufs[nxt % 2], sems[nxt % 2])
            descs[c % 2].wait()
            pltpu.sync_copy(bufs[c % 2],
                            out_hbm.at[pl.ds(base + c * _CH, _CH)])

    return k(x2d, flat_ids)


def _ffn_body(xg_ref, uw_ref, ub_ref, gw_ref, gb_ref, dw_ref, db_ref,
              wts_ref, ids_ref, out_ref, acc_ref, xbf_ref):
    e = pl.program_id(0)
    h = pl.program_id(1)
    dn = (((1,), (1,)), ((), ()))

    @pl.when(h == 0)
    def _():
        xbf_ref[...] = xg_ref[0].astype(jnp.bfloat16)

    xb = xbf_ref[...]
    uw = uw_ref[0].astype(jnp.bfloat16)
    gw = gw_ref[0].astype(jnp.bfloat16)
    up = lax.dot_general(xb, uw, dn,
                         preferred_element_type=jnp.float32) + ub_ref[0, 0]
    gt = lax.dot_general(xb, gw, dn,
                         preferred_element_type=jnp.float32) + gb_ref[0, 0]
    hb = (gt * jax.nn.sigmoid(gt) * up).astype(jnp.bfloat16)
    dw = dw_ref[0].astype(jnp.bfloat16)
    contrib = lax.dot_general(hb, dw, dn,
                              preferred_element_type=jnp.float32)

    @pl.when(h == 0)
    def _():
        acc_ref[...] = contrib

    @pl.when(h > 0)
    def _():
        acc_ref[...] += contrib

    @pl.when(h == NH - 1)
    def _():
        final = (acc_ref[...] + db_ref[0]) * wts_ref[0]
        tio = lax.broadcasted_iota(jnp.int32, (T, CAP), 0)
        oh = (tio == ids_ref[0]).astype(jnp.bfloat16)
        outc = lax.dot_general(oh, final.astype(jnp.bfloat16),
                               (((1,), (0,)), ((), ())),
                               preferred_element_type=jnp.float32)

        @pl.when(e == 0)
        def _():
            out_ref[...] = outc

        @pl.when(e > 0)
        def _():
            out_ref[...] += outc


def _ffn(xg4, up_w, up_b3, gateproj_w, gp_b3, down_w, db3, wts3, ids_row3):
    return pl.pallas_call(
        _ffn_body,
        grid=(E, NH),
        in_specs=[
            pl.BlockSpec((1, CAP, D), lambda e, h: (e, 0, 0)),
            pl.BlockSpec((1, BH, D), lambda e, h: (e, h, 0)),
            pl.BlockSpec((1, 1, 1, BH), lambda e, h: (e, h, 0, 0)),
            pl.BlockSpec((1, BH, D), lambda e, h: (e, h, 0)),
            pl.BlockSpec((1, 1, 1, BH), lambda e, h: (e, h, 0, 0)),
            pl.BlockSpec((1, D, BH), lambda e, h: (e, 0, h)),
            pl.BlockSpec((1, 1, D), lambda e, h: (e, 0, 0)),
            pl.BlockSpec((1, CAP, 1), lambda e, h: (e, 0, 0)),
            pl.BlockSpec((1, 1, CAP), lambda e, h: (e, 0, 0)),
        ],
        out_specs=pl.BlockSpec((T, D), lambda e, h: (0, 0)),
        out_shape=jax.ShapeDtypeStruct((T, D), jnp.float32),
        scratch_shapes=[pltpu.VMEM((CAP, D), jnp.float32),
                        pltpu.VMEM((CAP, D), jnp.bfloat16)],
    )(xg4, up_w, up_b3, gateproj_w, gp_b3, down_w, db3, wts3, ids_row3)


def kernel(hidden_states, gate_weight, up_w, up_b, gateproj_w, gateproj_b,
           down_w, down_b):
    b, s, d = hidden_states.shape
    x2d = hidden_states.reshape(b * s, d)
    scores, aux = _scores(x2d, gate_weight)
    sT3 = scores.T.reshape(E, NR, 512)
    ids3, wts3 = _topk(aux, sT3)
    flat_ids = ids3.reshape(E * CAP)
    xg = _gather(x2d, flat_ids)
    out = _ffn(
        xg.reshape(E, CAP, D),
        up_w, up_b.reshape(E, NH, 1, BH),
        gateproj_w, gateproj_b.reshape(E, NH, 1, BH),
        down_w, down_b.reshape(E, 1, D),
        wts3, ids3.reshape(E, 1, CAP),
    )
    return out.reshape(b, s, d)
